# baseline (speedup 1.0000x reference)
import functools

import jax
import jax.numpy as jnp
from jax import lax
from jax.experimental import pallas as pl
from jax.experimental.pallas import tpu as pltpu
from jax.experimental.pallas import tpu_sc as plsc

N = 50000
C = 128
NCOEF = 25
ROW = NCOEF * C
NW = 32
R = 32
S = 1568
CHUNKS = S // R
LAST = N - R

_mesh = plsc.VectorSubcoreMesh(core_axis_name="c", subcore_axis_name="s")


@functools.partial(
    pl.kernel,
    mesh=_mesh,
    out_type=jax.ShapeDtypeStruct((N, ROW), jnp.float32),
    scratch_types=[
        pltpu.VMEM((R,), jnp.int32),
        pltpu.VMEM((R, C), jnp.float32),
        pltpu.VMEM((R, ROW - C), jnp.float32),
        pltpu.SemaphoreType.DMA,
    ],
)
def _emb_sc(idx_hbm, table_hbm, zeros_hbm, out_hbm, idx_v, rows_v, zeros_v, sem):
    wid = lax.axis_index("s") * 2 + lax.axis_index("c")
    pltpu.sync_copy(zeros_hbm, zeros_v)

    def chunk(j, carry):
        base0 = wid * S + j * R

        @pl.when(base0 < N)
        def _():
            base = jnp.minimum(base0, LAST)
            pltpu.sync_copy(idx_hbm.at[pl.ds(base, R)], idx_v)
            pltpu.async_copy(table_hbm.at[idx_v], rows_v, sem).wait()
            pltpu.sync_copy(rows_v, out_hbm.at[pl.ds(base, R), pl.ds(0, C)])
            pltpu.sync_copy(zeros_v, out_hbm.at[pl.ds(base, R), pl.ds(C, ROW - C)])

        return carry

    lax.fori_loop(0, CHUNKS, chunk, 0)


def kernel(atomic_numbers, embedding_table):
    idx = atomic_numbers.astype(jnp.int32)
    zeros = jnp.zeros((R, ROW - C), jnp.float32)
    y = _emb_sc(idx, embedding_table, zeros)
    return y.reshape(N, NCOEF, C)


# speedup vs baseline: 1.0118x; 1.0118x over previous
import functools

import jax
import jax.numpy as jnp
from jax import lax
from jax.experimental import pallas as pl
from jax.experimental.pallas import tpu as pltpu
from jax.experimental.pallas import tpu_sc as plsc

N = 50000
C = 128
NCOEF = 25
ROW = NCOEF * C
NW = 32
R = 112
CHUNKS = 14
S = R * CHUNKS
LAST = N - R

_mesh = plsc.VectorSubcoreMesh(core_axis_name="c", subcore_axis_name="s")


@functools.partial(
    pl.kernel,
    mesh=_mesh,
    out_type=jax.ShapeDtypeStruct((N, ROW), jnp.float32),
    scratch_types=[
        pltpu.VMEM((2, R), jnp.int32),
        pltpu.VMEM((2, R, C), jnp.float32),
        pltpu.VMEM_SHARED((R, ROW - C), jnp.float32),
        pltpu.SemaphoreType.DMA,
        pltpu.SemaphoreType.DMA,
        pltpu.SemaphoreType.DMA,
        pltpu.SemaphoreType.DMA,
    ],
)
def _emb_sc(idx_hbm, table_hbm, zeros_hbm, out_hbm,
            idx_v, rows_v, zeros_s, sem_g, sem_w0, sem_w1, sem_z):
    cid = lax.axis_index("c")
    sid = lax.axis_index("s")
    wid = sid * 2 + cid

    @pl.when(sid == 0)
    def _():
        pltpu.sync_copy(zeros_hbm, zeros_s)

    plsc.subcore_barrier()

    sem_w = (sem_w0, sem_w1)
    for j in range(CHUNKS):
        b = j % 2
        base = jnp.minimum(wid * S + j * R, LAST)
        if j >= 2:
            pltpu.make_async_copy(
                rows_v.at[b], out_hbm.at[pl.ds(base, R), pl.ds(0, C)], sem_w[b]
            ).wait()
        pltpu.sync_copy(idx_hbm.at[pl.ds(base, R)], idx_v.at[b])
        pltpu.async_copy(table_hbm.at[idx_v.at[b]], rows_v.at[b], sem_g).wait()
        pltpu.async_copy(
            rows_v.at[b], out_hbm.at[pl.ds(base, R), pl.ds(0, C)], sem_w[b]
        )
        pltpu.async_copy(
            zeros_s, out_hbm.at[pl.ds(base, R), pl.ds(C, ROW - C)], sem_z
        )

    for j in (CHUNKS - 2, CHUNKS - 1):
        pltpu.make_async_copy(
            rows_v.at[j % 2], out_hbm.at[pl.ds(0, R), pl.ds(0, C)], sem_w[j % 2]
        ).wait()
    for _ in range(CHUNKS):
        pltpu.make_async_copy(
            zeros_s, out_hbm.at[pl.ds(0, R), pl.ds(C, ROW - C)], sem_z
        ).wait()


def kernel(atomic_numbers, embedding_table):
    idx = atomic_numbers.astype(jnp.int32)
    zeros = jnp.zeros((R, ROW - C), jnp.float32)
    y = _emb_sc(idx, embedding_table, zeros)
    return y.reshape(N, NCOEF, C)


# speedup vs baseline: 1.5595x; 1.5413x over previous
import functools

import jax
import jax.numpy as jnp
from jax import lax
from jax.experimental import pallas as pl
from jax.experimental.pallas import tpu as pltpu
from jax.experimental.pallas import tpu_sc as plsc

N = 50000
C = 128
NCOEF = 25
NW = 32
R = 8
CHUNKS = 196
GROUPS = CHUNKS // 2
S = R * CHUNKS
LAST = N - R

_mesh = plsc.VectorSubcoreMesh(core_axis_name="c", subcore_axis_name="s")


@functools.partial(
    pl.kernel,
    mesh=_mesh,
    out_type=jax.ShapeDtypeStruct((N, NCOEF, C), jnp.float32),
    scratch_types=[
        pltpu.VMEM((2, R), jnp.int32),
        pltpu.VMEM((2, R, NCOEF, C), jnp.float32),
        pltpu.SemaphoreType.DMA,
        pltpu.SemaphoreType.DMA,
        pltpu.SemaphoreType.DMA,
    ],
)
def _emb_sc(idx_hbm, table_hbm, zeros_hbm, out_hbm,
            idx_v, buf, sem_g, sem_w0, sem_w1):
    cid = lax.axis_index("c")
    sid = lax.axis_index("s")
    wid = sid * 2 + cid
    sem_w = (sem_w0, sem_w1)

    for b in range(2):
        pltpu.sync_copy(zeros_hbm, buf.at[b, :, pl.ds(1, NCOEF - 1), :])

    def group(g, carry):
        for b in range(2):
            base = jnp.minimum(wid * S + (2 * g + b) * R, LAST)

            @pl.when(g > 0)
            def _():
                pltpu.make_async_copy(
                    buf.at[b], out_hbm.at[pl.ds(base, R)], sem_w[b]
                ).wait()

            pltpu.sync_copy(idx_hbm.at[pl.ds(base, R)], idx_v.at[b])
            pltpu.async_copy(
                table_hbm.at[idx_v.at[b]], buf.at[b, :, pl.ds(0, 1), :], sem_g
            ).wait()
            pltpu.async_copy(buf.at[b], out_hbm.at[pl.ds(base, R)], sem_w[b])
        return carry

    lax.fori_loop(0, GROUPS, group, 0)

    for b in range(2):
        pltpu.make_async_copy(
            buf.at[b], out_hbm.at[pl.ds(0, R)], sem_w[b]
        ).wait()


def kernel(atomic_numbers, embedding_table):
    idx = atomic_numbers.astype(jnp.int32)
    table3 = embedding_table.reshape(embedding_table.shape[0], 1, C)
    zeros = jnp.zeros((R, NCOEF - 1, C), jnp.float32)
    return _emb_sc(idx, table3, zeros)


# speedup vs baseline: 1.6142x; 1.0350x over previous
import functools

import jax
import jax.numpy as jnp
from jax import lax
from jax.experimental import pallas as pl
from jax.experimental.pallas import tpu as pltpu
from jax.experimental.pallas import tpu_sc as plsc

N = 50000
C = 128
NCOEF = 25
NZ = NCOEF - 1
NW = 32
S = 1584
R = 8
CH = S // R
NB = 3
GROUPS = CH // NB
LA = 1

_mesh = plsc.VectorSubcoreMesh(core_axis_name="c", subcore_axis_name="s")


@functools.partial(
    pl.kernel,
    mesh=_mesh,
    out_type=jax.ShapeDtypeStruct((N, NCOEF, C), jnp.float32),
    scratch_types=[
        pltpu.VMEM((S,), jnp.int32),
        pltpu.VMEM((NB, R, NCOEF, C), jnp.float32),
        pltpu.SemaphoreType.DMA,
        pltpu.SemaphoreType.DMA,
        pltpu.SemaphoreType.DMA,
        pltpu.SemaphoreType.DMA,
        pltpu.SemaphoreType.DMA,
        pltpu.SemaphoreType.DMA,
    ],
)
def _emb_sc(idx_hbm, table_hbm, zeros_hbm, out_hbm, idx_v, buf,
            gsem0, gsem1, gsem2, wsem0, wsem1, wsem2):
    cid = lax.axis_index("c")
    sid = lax.axis_index("s")
    wid = sid * 2 + cid
    gsem = (gsem0, gsem1, gsem2)
    wsem = (wsem0, wsem1, wsem2)
    base_w = jnp.minimum(wid * S, N - S)

    for b in range(NB):
        pltpu.sync_copy(zeros_hbm, buf.at[b, :, pl.ds(1, NZ), :])

    pltpu.sync_copy(idx_hbm.at[pl.ds(base_w, S)], idx_v)

    for b in range(LA):
        pltpu.async_copy(
            table_hbm.at[idx_v.at[pl.ds(b * R, R)]],
            buf.at[b, :, pl.ds(0, 1), :], gsem[b],
        )

    def group(g, carry):
        for b in range(NB):
            c = NB * g + b
            pltpu.make_async_copy(
                table_hbm.at[idx_v.at[pl.ds(0, R)]],
                buf.at[b, :, pl.ds(0, 1), :], gsem[b],
            ).wait()
            pltpu.async_copy(
                buf.at[b], out_hbm.at[pl.ds(base_w + c * R, R)], wsem[b]
            )

            bn = (b + LA) % NB

            @pl.when(jnp.logical_and(c >= NB - LA, c + LA < CH))
            def _():
                pltpu.make_async_copy(
                    buf.at[bn], out_hbm.at[pl.ds(0, R)], wsem[bn]
                ).wait()

            @pl.when(c + LA < CH)
            def _():
                pltpu.async_copy(
                    table_hbm.at[idx_v.at[pl.ds((c + LA) * R, R)]],
                    buf.at[bn, :, pl.ds(0, 1), :], gsem[bn],
                )
        return carry

    lax.fori_loop(0, GROUPS, group, 0)

    for b in range(NB):
        pltpu.make_async_copy(
            buf.at[b], out_hbm.at[pl.ds(0, R)], wsem[b]
        ).wait()


def kernel(atomic_numbers, embedding_table):
    idx = atomic_numbers.astype(jnp.int32)
    table3 = embedding_table.reshape(embedding_table.shape[0], 1, C)
    zeros = jnp.zeros((R, NZ, C), jnp.float32)
    return _emb_sc(idx, table3, zeros)


# speedup vs baseline: 1.8539x; 1.1485x over previous
import functools

import jax
import jax.numpy as jnp
from jax import lax
from jax.experimental import pallas as pl
from jax.experimental.pallas import tpu as pltpu
from jax.experimental.pallas import tpu_sc as plsc

N = 50000
C = 128
NCOEF = 25
NZ = NCOEF - 1
NW = 32
S = 1584
R = 8
CH = S // R
NB = 3
GROUPS = CH // NB
LA = 1

_mesh = plsc.VectorSubcoreMesh(core_axis_name="c", subcore_axis_name="s")


@functools.partial(
    pl.kernel,
    mesh=_mesh,
    out_type=jax.ShapeDtypeStruct((N, NCOEF, C), jnp.float32),
    scratch_types=[
        pltpu.VMEM((S,), jnp.int32),
        pltpu.VMEM((NB, R, NCOEF, C), jnp.float32),
        pltpu.VMEM_SHARED((100, 1, C), jnp.float32),
        pltpu.SemaphoreType.DMA,
        pltpu.SemaphoreType.DMA,
        pltpu.SemaphoreType.DMA,
        pltpu.SemaphoreType.DMA,
        pltpu.SemaphoreType.DMA,
        pltpu.SemaphoreType.DMA,
    ],
)
def _emb_sc(idx_hbm, table_hbm, zeros_hbm, out_hbm, idx_v, buf, table_s,
            gsem0, gsem1, gsem2, wsem0, wsem1, wsem2):
    cid = lax.axis_index("c")
    sid = lax.axis_index("s")
    wid = sid * 2 + cid
    gsem = (gsem0, gsem1, gsem2)
    wsem = (wsem0, wsem1, wsem2)
    base_w = jnp.minimum(wid * S, N - S)

    for b in range(NB):
        pltpu.sync_copy(zeros_hbm, buf.at[b, :, pl.ds(1, NZ), :])

    pltpu.sync_copy(idx_hbm.at[pl.ds(base_w, S)], idx_v)

    @pl.when(sid == 0)
    def _():
        pltpu.sync_copy(table_hbm, table_s)

    plsc.subcore_barrier()

    for b in range(LA):
        pltpu.async_copy(
            table_s.at[idx_v.at[pl.ds(b * R, R)]],
            buf.at[b, :, pl.ds(0, 1), :], gsem[b],
        )

    def group(g, carry):
        for b in range(NB):
            c = NB * g + b
            pltpu.make_async_copy(
                table_s.at[idx_v.at[pl.ds(0, R)]],
                buf.at[b, :, pl.ds(0, 1), :], gsem[b],
            ).wait()
            pltpu.async_copy(
                buf.at[b], out_hbm.at[pl.ds(base_w + c * R, R)], wsem[b]
            )

            bn = (b + LA) % NB

            @pl.when(jnp.logical_and(c >= NB - LA, c + LA < CH))
            def _():
                pltpu.make_async_copy(
                    buf.at[bn], out_hbm.at[pl.ds(0, R)], wsem[bn]
                ).wait()

            @pl.when(c + LA < CH)
            def _():
                pltpu.async_copy(
                    table_s.at[idx_v.at[pl.ds((c + LA) * R, R)]],
                    buf.at[bn, :, pl.ds(0, 1), :], gsem[bn],
                )
        return carry

    lax.fori_loop(0, GROUPS, group, 0)

    for b in range(NB):
        pltpu.make_async_copy(
            buf.at[b], out_hbm.at[pl.ds(0, R)], wsem[b]
        ).wait()


def kernel(atomic_numbers, embedding_table):
    idx = atomic_numbers.astype(jnp.int32)
    table3 = embedding_table.reshape(embedding_table.shape[0], 1, C)
    zeros = jnp.zeros((R, NZ, C), jnp.float32)
    return _emb_sc(idx, table3, zeros)
